# Initial kernel scaffold; baseline (speedup 1.0000x reference)
import jax
import jax.numpy as jnp
from jax.experimental import pallas as pl


def kernel(scores, labels):
    raise NotImplementedError("write your pallas kernel here")


# speedup vs baseline: 1.7034x; 1.7034x over previous
import functools

import jax
import jax.numpy as jnp
from jax import lax
from jax.experimental import pallas as pl
from jax.experimental.pallas import tpu as pltpu
from jax.experimental.pallas import tpu_sc as plsc

_TOPK = 5
_ROWS = 128
_COLS = 32768
_LANES = 16
_CHUNKS_PER_ROW = _COLS // _LANES
_GATHER_WORKERS = 8
_BLK = 2048
_NBLK = _COLS // _BLK


def _sc_gather_body(table, flat_idx, out, idx_v, val_v, sem):
    wid = lax.axis_index("s") * 2 + lax.axis_index("c")

    @pl.when(wid < _GATHER_WORKERS)
    def _():
        base = wid * _LANES
        pltpu.sync_copy(flat_idx.at[pl.ds(base, _LANES)], idx_v)
        pltpu.async_copy(table.at[idx_v], val_v, sem).wait()
        pltpu.sync_copy(val_v, out.at[pl.ds(base, _LANES)])


def _make_sc_gather():
    return functools.partial(
        pl.kernel,
        mesh=plsc.VectorSubcoreMesh(core_axis_name="c", subcore_axis_name="s"),
        out_type=jax.ShapeDtypeStruct((_ROWS,), jnp.float32),
        scratch_types=[
            pltpu.VMEM((_LANES,), jnp.int32),
            pltpu.VMEM((_LANES,), jnp.float32),
            pltpu.SemaphoreType.DMA,
        ],
    )(_sc_gather_body)


def _count_body(v_ref, lab_ref, s_ref, out_ref, acc_ref):
    j = pl.program_id(0)

    @pl.when(j == 0)
    def _():
        acc_ref[...] = jnp.zeros_like(acc_ref)

    s = s_ref[...]
    v = v_ref[...]
    lab = lab_ref[...]
    col = lax.broadcasted_iota(jnp.int32, s.shape, 1) + j * _BLK
    ahead = (s > v) | ((s == v) & (col < lab))
    acc_ref[...] += jnp.sum(ahead.astype(jnp.int32), axis=1, keepdims=True)

    @pl.when(j == _NBLK - 1)
    def _():
        hits = (acc_ref[...] < _TOPK).astype(jnp.float32)
        out_ref[...] = (jnp.sum(hits) / _ROWS).reshape(1, 1)


_count_call = pl.pallas_call(
    _count_body,
    grid=(_NBLK,),
    in_specs=[
        pl.BlockSpec((_ROWS, 1), lambda j: (0, 0)),
        pl.BlockSpec((_ROWS, 1), lambda j: (0, 0)),
        pl.BlockSpec((_ROWS, _BLK), lambda j: (0, j)),
    ],
    out_specs=pl.BlockSpec((1, 1), lambda j: (0, 0)),
    out_shape=jax.ShapeDtypeStruct((1, 1), jnp.float32),
    scratch_shapes=[pltpu.VMEM((_ROWS, 1), jnp.int32)],
)


def kernel(scores, labels):
    labels = labels.astype(jnp.int32)
    flat_idx = jnp.arange(_ROWS, dtype=jnp.int32) * _COLS + labels
    table = scores.reshape(_ROWS * _COLS)
    v = _make_sc_gather()(table, flat_idx)
    out = _count_call(v.reshape(_ROWS, 1), labels.reshape(_ROWS, 1), scores)
    return out[0, 0]
